# baseline (speedup 1.0000x reference)
import functools

import jax
import jax.numpy as jnp
from jax.experimental import pallas as pl
from jax.experimental.pallas import tpu as pltpu

BULK = 1024
CH = 128


def _tail_copy(k_hbm, v_hbm, ktail, vtail, sem, b, layer, buf, c, start):
    kcp = pltpu.make_async_copy(
        k_hbm.at[b, layer, :, pl.ds(BULK + c * CH, CH), :],
        ktail.at[buf, :, pl.ds(c * CH, CH), :],
        sem.at[buf])
    vcp = pltpu.make_async_copy(
        v_hbm.at[b, layer, :, pl.ds(BULK + c * CH, CH), :],
        vtail.at[buf, :, pl.ds(c * CH, CH), :],
        sem.at[buf])
    if start:
        kcp.start()
        vcp.start()
    else:
        kcp.wait()
        vcp.wait()


def _attn_kernel(ctx_ref, q_ref, kb_ref, vb_ref, k_hbm, v_hbm, o_ref,
                 ktail, vtail, qs_ref, m_ref, l_ref, acc_ref, sem, *,
                 scale, num_layers, batch, kvh, g):
    b = pl.program_id(0)
    layer = pl.program_id(1)
    pair = b * num_layers + layer
    n_pairs = batch * num_layers
    buf = jax.lax.rem(pair, 2)
    ctx = ctx_ref[b]

    def ntail(c):
        return jnp.maximum(jax.lax.div(c - BULK + (CH - 1), CH), 0)

    def tail_io(p, tbuf, start):
        pb = jax.lax.div(p, num_layers)
        pl_ = jax.lax.rem(p, num_layers)
        nc = ntail(ctx_ref[pb])

        def body(c, _):
            _tail_copy(k_hbm, v_hbm, ktail, vtail, sem, pb, pl_, tbuf, c,
                       start)
            return 0
        jax.lax.fori_loop(0, nc, body, 0)

    @pl.when(pair == 0)
    def _prologue():
        tail_io(0, 0, True)

    @pl.when(pair + 1 < n_pairs)
    def _prefetch_tail():
        tail_io(pair + 1, 1 - buf, True)

    @pl.when(layer == 0)
    def _load_q():
        qs_ref[...] = q_ref[0] * scale

    q = qs_ref[...]

    kb = kb_ref[0, 0]
    vb = vb_ref[0, 0]
    s = jax.lax.dot_general(
        q, kb, (((2,), (2,)), ((0,), (0,))),
        preferred_element_type=jnp.float32)
    pos = jax.lax.broadcasted_iota(jnp.int32, (kvh, g, BULK), 2)
    s = jnp.where(pos < ctx, s, -1e30)
    m0 = jnp.max(s, axis=2, keepdims=True)
    p = jnp.exp(s - m0)
    m_ref[...] = jnp.broadcast_to(m0, m_ref.shape)
    l_ref[...] = jnp.broadcast_to(
        jnp.sum(p, axis=2, keepdims=True), l_ref.shape)
    acc_ref[...] = jax.lax.dot_general(
        p, vb, (((2,), (1,)), ((0,), (0,))),
        preferred_element_type=jnp.float32)

    tail_io(pair, buf, False)

    def chunk_step(c, _):
        k = ktail[buf, :, pl.ds(c * CH, CH), :]
        v = vtail[buf, :, pl.ds(c * CH, CH), :]
        s = jax.lax.dot_general(
            q, k, (((2,), (2,)), ((0,), (0,))),
            preferred_element_type=jnp.float32)
        pos = BULK + c * CH + jax.lax.broadcasted_iota(
            jnp.int32, (kvh, g, CH), 2)
        s = jnp.where(pos < ctx, s, -1e30)

        m_prev = m_ref[...]
        s_max = jnp.max(s, axis=2, keepdims=True)
        m_new = jnp.maximum(m_prev, s_max)
        alpha = jnp.exp(m_prev - m_new)
        p = jnp.exp(s - m_new[:, :, :1])
        l_ref[...] = l_ref[...] * alpha + jnp.sum(p, axis=2, keepdims=True)
        acc_ref[...] = acc_ref[...] * alpha + jax.lax.dot_general(
            p, v, (((2,), (1,)), ((0,), (0,))),
            preferred_element_type=jnp.float32)
        m_ref[...] = m_new
        return 0

    jax.lax.fori_loop(0, ntail(ctx), chunk_step, 0)

    out = acc_ref[...] / l_ref[...]

    @pl.when(layer == num_layers - 1)
    def _write_out():
        o_ref[0] = out

    @pl.when(layer < num_layers - 1)
    def _carry_q():
        qs_ref[...] = out * scale


@jax.jit
def kernel(query, k_cache, v_cache, context_lens):
    B, H, D = query.shape
    L = k_cache.shape[1]
    KVH = k_cache.shape[2]
    S = k_cache.shape[3]
    G = H // KVH
    scale = 1.0 / D ** 0.5

    q4 = query.reshape(B, KVH, G, D)

    def q_map(b, layer, ctx):
        return (b, 0, 0, 0)

    def kv_map(b, layer, ctx):
        return (b, layer, 0, 0, 0)

    grid_spec = pltpu.PrefetchScalarGridSpec(
        num_scalar_prefetch=1,
        grid=(B, L),
        in_specs=[
            pl.BlockSpec((1, KVH, G, D), q_map),
            pl.BlockSpec((1, 1, KVH, BULK, D), kv_map),
            pl.BlockSpec((1, 1, KVH, BULK, D), kv_map),
            pl.BlockSpec(memory_space=pltpu.MemorySpace.HBM),
            pl.BlockSpec(memory_space=pltpu.MemorySpace.HBM),
        ],
        out_specs=pl.BlockSpec((1, KVH, G, D), q_map),
        scratch_shapes=[
            pltpu.VMEM((2, KVH, S - BULK, D), jnp.float32),
            pltpu.VMEM((2, KVH, S - BULK, D), jnp.float32),
            pltpu.VMEM((KVH, G, D), jnp.float32),
            pltpu.VMEM((KVH, G, 128), jnp.float32),
            pltpu.VMEM((KVH, G, 128), jnp.float32),
            pltpu.VMEM((KVH, G, D), jnp.float32),
            pltpu.SemaphoreType.DMA((2,)),
        ],
    )
    out = pl.pallas_call(
        functools.partial(_attn_kernel, scale=scale, num_layers=L,
                          batch=B, kvh=KVH, g=G),
        grid_spec=grid_spec,
        out_shape=jax.ShapeDtypeStruct((B, KVH, G, D), jnp.float32),
        compiler_params=pltpu.CompilerParams(
            dimension_semantics=("arbitrary", "arbitrary"),
            vmem_limit_bytes=100 * 1024 * 1024),
    )(context_lens, q4, k_cache, v_cache, k_cache, v_cache)
    return out.reshape(B, H, D)


# speedup vs baseline: 1.0703x; 1.0703x over previous
import functools

import jax
import jax.numpy as jnp
from jax.experimental import pallas as pl
from jax.experimental.pallas import tpu as pltpu

BULK = 1024
CH = 256


def _tail_copy(k_hbm, v_hbm, ktail, vtail, sem, b, layer, buf, c, start):
    kcp = pltpu.make_async_copy(
        k_hbm.at[b, layer, :, pl.ds(BULK + c * CH, CH), :],
        ktail.at[buf, :, pl.ds(c * CH, CH), :],
        sem.at[buf])
    vcp = pltpu.make_async_copy(
        v_hbm.at[b, layer, :, pl.ds(BULK + c * CH, CH), :],
        vtail.at[buf, :, pl.ds(c * CH, CH), :],
        sem.at[buf])
    if start:
        kcp.start()
        vcp.start()
    else:
        kcp.wait()
        vcp.wait()


def _attn_kernel(ctx_ref, q_ref, kb_ref, vb_ref, k_hbm, v_hbm, o_ref,
                 ktail, vtail, qs_ref, m_ref, l_ref, acc_ref, sem, *,
                 scale, num_layers, batch, kvh, g):
    b = pl.program_id(0)
    layer = pl.program_id(1)
    pair = b * num_layers + layer
    n_pairs = batch * num_layers
    buf = jax.lax.rem(pair, 2)
    ctx = ctx_ref[b]

    def ntail(c):
        return jnp.maximum(jax.lax.div(c - BULK + (CH - 1), CH), 0)

    def tail_io(p, tbuf, start):
        pb = jax.lax.div(p, num_layers)
        pl_ = jax.lax.rem(p, num_layers)
        nc = ntail(ctx_ref[pb])

        def body(c, _):
            _tail_copy(k_hbm, v_hbm, ktail, vtail, sem, pb, pl_, tbuf, c,
                       start)
            return 0
        jax.lax.fori_loop(0, nc, body, 0)

    @pl.when(pair == 0)
    def _prologue():
        tail_io(0, 0, True)

    @pl.when(pair + 1 < n_pairs)
    def _prefetch_tail():
        tail_io(pair + 1, 1 - buf, True)

    @pl.when(layer == 0)
    def _load_q():
        qs_ref[...] = q_ref[0] * scale

    q = qs_ref[...]

    kb = kb_ref[0, 0]
    vb = vb_ref[0, 0]
    s = jax.lax.dot_general(
        q, kb, (((2,), (2,)), ((0,), (0,))),
        preferred_element_type=jnp.float32)
    pos = jax.lax.broadcasted_iota(jnp.int32, (kvh, g, BULK), 2)
    s = jnp.where(pos < ctx, s, -1e30)
    m0 = jnp.max(s, axis=2, keepdims=True)
    p = jnp.exp(s - m0)
    m_ref[...] = jnp.broadcast_to(m0, m_ref.shape)
    l_ref[...] = jnp.broadcast_to(
        jnp.sum(p, axis=2, keepdims=True), l_ref.shape)
    acc_ref[...] = jax.lax.dot_general(
        p, vb, (((2,), (1,)), ((0,), (0,))),
        preferred_element_type=jnp.float32)

    def chunk_step(c, _):
        _tail_copy(k_hbm, v_hbm, ktail, vtail, sem, b, layer, buf, c, False)
        k = ktail[buf, :, pl.ds(c * CH, CH), :]
        v = vtail[buf, :, pl.ds(c * CH, CH), :]
        s = jax.lax.dot_general(
            q, k, (((2,), (2,)), ((0,), (0,))),
            preferred_element_type=jnp.float32)
        pos = BULK + c * CH + jax.lax.broadcasted_iota(
            jnp.int32, (kvh, g, CH), 2)
        s = jnp.where(pos < ctx, s, -1e30)

        m_prev = m_ref[...]
        s_max = jnp.max(s, axis=2, keepdims=True)
        m_new = jnp.maximum(m_prev, s_max)
        alpha = jnp.exp(m_prev - m_new)
        p = jnp.exp(s - m_new[:, :, :1])
        l_ref[...] = l_ref[...] * alpha + jnp.sum(p, axis=2, keepdims=True)
        acc_ref[...] = acc_ref[...] * alpha + jax.lax.dot_general(
            p, v, (((2,), (1,)), ((0,), (0,))),
            preferred_element_type=jnp.float32)
        m_ref[...] = m_new
        return 0

    jax.lax.fori_loop(0, ntail(ctx), chunk_step, 0)

    out = acc_ref[...] / l_ref[...]

    @pl.when(layer == num_layers - 1)
    def _write_out():
        o_ref[0] = out

    @pl.when(layer < num_layers - 1)
    def _carry_q():
        qs_ref[...] = out * scale


@jax.jit
def kernel(query, k_cache, v_cache, context_lens):
    B, H, D = query.shape
    L = k_cache.shape[1]
    KVH = k_cache.shape[2]
    S = k_cache.shape[3]
    G = H // KVH
    scale = 1.0 / D ** 0.5

    q4 = query.reshape(B, KVH, G, D)

    def q_map(b, layer, ctx):
        return (b, 0, 0, 0)

    def kv_map(b, layer, ctx):
        return (b, layer, 0, 0, 0)

    grid_spec = pltpu.PrefetchScalarGridSpec(
        num_scalar_prefetch=1,
        grid=(B, L),
        in_specs=[
            pl.BlockSpec((1, KVH, G, D), q_map),
            pl.BlockSpec((1, 1, KVH, BULK, D), kv_map),
            pl.BlockSpec((1, 1, KVH, BULK, D), kv_map),
            pl.BlockSpec(memory_space=pltpu.MemorySpace.HBM),
            pl.BlockSpec(memory_space=pltpu.MemorySpace.HBM),
        ],
        out_specs=pl.BlockSpec((1, KVH, G, D), q_map),
        scratch_shapes=[
            pltpu.VMEM((2, KVH, S - BULK, D), jnp.float32),
            pltpu.VMEM((2, KVH, S - BULK, D), jnp.float32),
            pltpu.VMEM((KVH, G, D), jnp.float32),
            pltpu.VMEM((KVH, G, 128), jnp.float32),
            pltpu.VMEM((KVH, G, 128), jnp.float32),
            pltpu.VMEM((KVH, G, D), jnp.float32),
            pltpu.SemaphoreType.DMA((2,)),
        ],
    )
    out = pl.pallas_call(
        functools.partial(_attn_kernel, scale=scale, num_layers=L,
                          batch=B, kvh=KVH, g=G),
        grid_spec=grid_spec,
        out_shape=jax.ShapeDtypeStruct((B, KVH, G, D), jnp.float32),
        compiler_params=pltpu.CompilerParams(
            dimension_semantics=("arbitrary", "arbitrary"),
            vmem_limit_bytes=100 * 1024 * 1024),
    )(context_lens, q4, k_cache, v_cache, k_cache, v_cache)
    return out.reshape(B, H, D)


# speedup vs baseline: 1.0711x; 1.0007x over previous
import functools

import jax
import jax.numpy as jnp
from jax.experimental import pallas as pl
from jax.experimental.pallas import tpu as pltpu

BULK = 1024
CH = 256


def _tail_copy(k_hbm, v_hbm, ktail, vtail, sem, b, layer, buf, c, start):
    kcp = pltpu.make_async_copy(
        k_hbm.at[b, layer, :, pl.ds(BULK + c * CH, CH), :],
        ktail.at[buf, :, pl.ds(c * CH, CH), :],
        sem.at[buf])
    vcp = pltpu.make_async_copy(
        v_hbm.at[b, layer, :, pl.ds(BULK + c * CH, CH), :],
        vtail.at[buf, :, pl.ds(c * CH, CH), :],
        sem.at[buf])
    if start:
        kcp.start()
        vcp.start()
    else:
        kcp.wait()
        vcp.wait()


def _attn_kernel(ctx_ref, q_ref, kb_ref, vb_ref, k_hbm, v_hbm, o_ref,
                 ktail, vtail, qs_ref, m_ref, l_ref, acc_ref, sem, *,
                 scale, num_layers, batch, kvh, g, max_tail):
    b = pl.program_id(0)
    layer = pl.program_id(1)
    pair = b * num_layers + layer
    n_pairs = batch * num_layers
    buf = jax.lax.rem(pair, 2)
    ctx = ctx_ref[b]

    def ntail(c):
        return jnp.maximum(jax.lax.div(c - BULK + (CH - 1), CH), 0)

    def tail_io(p, tbuf, start):
        pb = jax.lax.div(p, num_layers)
        pl_ = jax.lax.rem(p, num_layers)
        nc = ntail(ctx_ref[pb])
        for c in range(max_tail):
            @pl.when(c < nc)
            def _():
                _tail_copy(k_hbm, v_hbm, ktail, vtail, sem, pb, pl_, tbuf, c,
                           start)

    @pl.when(pair == 0)
    def _prologue():
        tail_io(0, 0, True)

    @pl.when(pair + 1 < n_pairs)
    def _prefetch_tail():
        tail_io(pair + 1, 1 - buf, True)

    @pl.when(layer == 0)
    def _load_q():
        qs_ref[...] = q_ref[0] * scale

    q = qs_ref[...]

    kb = kb_ref[0, 0]
    vb = vb_ref[0, 0]
    s = jax.lax.dot_general(
        q, kb, (((2,), (2,)), ((0,), (0,))),
        preferred_element_type=jnp.float32)
    pos = jax.lax.broadcasted_iota(jnp.int32, (kvh, g, BULK), 2)
    s = jnp.where(pos < ctx, s, -1e30)
    m0 = jnp.max(s, axis=2, keepdims=True)
    p = jnp.exp(s - m0)
    m_ref[...] = jnp.broadcast_to(m0, m_ref.shape)
    l_ref[...] = jnp.broadcast_to(
        jnp.sum(p, axis=2, keepdims=True), l_ref.shape)
    acc_ref[...] = jax.lax.dot_general(
        p, vb, (((2,), (1,)), ((0,), (0,))),
        preferred_element_type=jnp.float32)

    def chunk_step(c, _):
        _tail_copy(k_hbm, v_hbm, ktail, vtail, sem, b, layer, buf, c, False)
        k = ktail[buf, :, pl.ds(c * CH, CH), :]
        v = vtail[buf, :, pl.ds(c * CH, CH), :]
        s = jax.lax.dot_general(
            q, k, (((2,), (2,)), ((0,), (0,))),
            preferred_element_type=jnp.float32)
        pos = BULK + c * CH + jax.lax.broadcasted_iota(
            jnp.int32, (kvh, g, CH), 2)
        s = jnp.where(pos < ctx, s, -1e30)

        m_prev = m_ref[...]
        s_max = jnp.max(s, axis=2, keepdims=True)
        m_new = jnp.maximum(m_prev, s_max)
        alpha = jnp.exp(m_prev - m_new)
        p = jnp.exp(s - m_new[:, :, :1])
        l_ref[...] = l_ref[...] * alpha + jnp.sum(p, axis=2, keepdims=True)
        acc_ref[...] = acc_ref[...] * alpha + jax.lax.dot_general(
            p, v, (((2,), (1,)), ((0,), (0,))),
            preferred_element_type=jnp.float32)
        m_ref[...] = m_new
        return 0

    jax.lax.fori_loop(0, ntail(ctx), chunk_step, 0)

    out = acc_ref[...] / l_ref[...]

    @pl.when(layer == num_layers - 1)
    def _write_out():
        o_ref[0] = out

    @pl.when(layer < num_layers - 1)
    def _carry_q():
        qs_ref[...] = out * scale


@jax.jit
def kernel(query, k_cache, v_cache, context_lens):
    B, H, D = query.shape
    L = k_cache.shape[1]
    KVH = k_cache.shape[2]
    S = k_cache.shape[3]
    G = H // KVH
    scale = 1.0 / D ** 0.5

    q4 = query.reshape(B, KVH, G, D)

    def q_map(b, layer, ctx):
        return (b, 0, 0, 0)

    def kv_map(b, layer, ctx):
        return (b, layer, 0, 0, 0)

    grid_spec = pltpu.PrefetchScalarGridSpec(
        num_scalar_prefetch=1,
        grid=(B, L),
        in_specs=[
            pl.BlockSpec((1, KVH, G, D), q_map),
            pl.BlockSpec((1, 1, KVH, BULK, D), kv_map),
            pl.BlockSpec((1, 1, KVH, BULK, D), kv_map),
            pl.BlockSpec(memory_space=pltpu.MemorySpace.HBM),
            pl.BlockSpec(memory_space=pltpu.MemorySpace.HBM),
        ],
        out_specs=pl.BlockSpec((1, KVH, G, D), q_map),
        scratch_shapes=[
            pltpu.VMEM((2, KVH, S - BULK, D), jnp.float32),
            pltpu.VMEM((2, KVH, S - BULK, D), jnp.float32),
            pltpu.VMEM((KVH, G, D), jnp.float32),
            pltpu.VMEM((KVH, G, 128), jnp.float32),
            pltpu.VMEM((KVH, G, 128), jnp.float32),
            pltpu.VMEM((KVH, G, D), jnp.float32),
            pltpu.SemaphoreType.DMA((2,)),
        ],
    )
    out = pl.pallas_call(
        functools.partial(_attn_kernel, scale=scale, num_layers=L,
                          batch=B, kvh=KVH, g=G,
                          max_tail=(S - BULK) // CH),
        grid_spec=grid_spec,
        out_shape=jax.ShapeDtypeStruct((B, KVH, G, D), jnp.float32),
        compiler_params=pltpu.CompilerParams(
            dimension_semantics=("arbitrary", "arbitrary"),
            vmem_limit_bytes=100 * 1024 * 1024),
    )(context_lens, q4, k_cache, v_cache, k_cache, v_cache)
    return out.reshape(B, H, D)
